# baseline (device time: 16592 ns/iter reference)
import jax
import jax.numpy as jnp
from jax import lax
from jax.experimental import pallas as pl
from jax.experimental.pallas import tpu as pltpu

N_DEV = 4
M = 1024
D = 256
H = 512
N_EXP = 16
N_EXP_LOCAL = N_EXP // N_DEV
CAP = 51
CHUNK = M // N_DEV
HALF = CHUNK // 2
DST_OFFSETS = (2, 1, 3)


def kernel(x, router_W, route_idx, expert_W):
    del router_W

    def body(x_ref, idx_ref, w_ref, out_ref,
             gate_ref, send_buf, recv_buf, send_sems, recv_sems):
        my_i = lax.axis_index("i")

        barrier = pltpu.get_barrier_semaphore()
        for k in range(1, N_DEV):
            peer = lax.rem(my_i + k, N_DEV)
            pl.semaphore_signal(
                barrier, inc=1,
                device_id=(peer,), device_id_type=pl.DeviceIdType.MESH,
            )
        pl.semaphore_wait(barrier, N_DEV - 1)

        route = idx_ref[:, :]
        e_ids = lax.broadcasted_iota(jnp.int32, (CHUNK, N_EXP), 1)
        le_ids = lax.broadcasted_iota(jnp.int32, (CHUNK, N_EXP_LOCAL), 1)
        ri = lax.broadcasted_iota(jnp.int32, (CHUNK, CHUNK), 0)
        ci = lax.broadcasted_iota(jnp.int32, (CHUNK, CHUNK), 1)
        tril = (ci <= ri).astype(jnp.bfloat16)

        offset = jnp.zeros((1, N_EXP), jnp.float32)
        for c in range(N_DEV):
            route_c = route[c * CHUNK:(c + 1) * CHUNK, :]
            oh_c = (route_c == e_ids).astype(jnp.bfloat16)
            cum_c = jnp.dot(tril, oh_c,
                            preferred_element_type=jnp.float32) + offset
            offset = offset + jnp.sum(oh_c.astype(jnp.float32), axis=0,
                                      keepdims=True)
            keep_c = (oh_c.astype(jnp.float32)
                      * (cum_c <= CAP).astype(jnp.float32))
            kept_c = jnp.sum(keep_c, axis=1, keepdims=True)
            local_c = route_c - my_i * N_EXP_LOCAL
            gate_ref[c * CHUNK:(c + 1) * CHUNK, :] = (
                (local_c == le_ids).astype(jnp.float32) * kept_c
            ).astype(jnp.bfloat16)

        w_cat = jnp.reshape(w_ref[:, :, :],
                            (N_EXP_LOCAL * D, H)).astype(jnp.bfloat16)

        def slab_out(base):
            x_s = x_ref[pl.ds(base, HALF), :].astype(jnp.bfloat16)
            g = gate_ref[pl.ds(base, HALF), :]
            xg = jnp.concatenate(
                [x_s * g[:, le:le + 1] for le in range(N_EXP_LOCAL)], axis=1)
            return jnp.dot(xg, w_cat, preferred_element_type=jnp.float32)

        rdmas = []
        for k, off in enumerate(DST_OFFSETS):
            dst = lax.rem(my_i + off, N_DEV)
            pair = 3 - off
            for h in range(2):
                sb = 2 * k + h
                rb = 2 * pair + h
                send_buf[sb, :, :] = slab_out(
                    dst * CHUNK + h * HALF).astype(jnp.bfloat16)
                rdma = pltpu.make_async_remote_copy(
                    src_ref=send_buf.at[sb],
                    dst_ref=recv_buf.at[rb],
                    send_sem=send_sems.at[sb],
                    recv_sem=recv_sems.at[rb],
                    device_id=(dst,),
                    device_id_type=pl.DeviceIdType.MESH,
                )
                rdma.start()
                rdmas.append(rdma)

        own = [slab_out(my_i * CHUNK + h * HALF) for h in range(2)]

        for rdma in rdmas:
            rdma.wait_send()
        for rdma in rdmas:
            rdma.wait_recv()

        for h in range(2):
            out_ref[pl.ds(h * HALF, HALF), :] = (
                own[h]
                + recv_buf[0 + h, :, :].astype(jnp.float32)
                + recv_buf[2 + h, :, :].astype(jnp.float32)
                + recv_buf[4 + h, :, :].astype(jnp.float32))

    return pl.pallas_call(
        body,
        out_shape=jax.ShapeDtypeStruct((CHUNK, H), jnp.float32),
        in_specs=[
            pl.BlockSpec(memory_space=pltpu.VMEM),
            pl.BlockSpec(memory_space=pltpu.VMEM),
            pl.BlockSpec(memory_space=pltpu.VMEM),
        ],
        out_specs=pl.BlockSpec(memory_space=pltpu.VMEM),
        scratch_shapes=[
            pltpu.VMEM((M, N_EXP_LOCAL), jnp.bfloat16),
            pltpu.VMEM((6, HALF, H), jnp.bfloat16),
            pltpu.VMEM((6, HALF, H), jnp.bfloat16),
            pltpu.SemaphoreType.DMA((6,)),
            pltpu.SemaphoreType.DMA((6,)),
        ],
        compiler_params=pltpu.CompilerParams(collective_id=0),
    )(x, route_idx, expert_W)
